# baseline (speedup 1.0000x reference)
import functools

import jax
import jax.numpy as jnp
from jax import lax
from jax.experimental import pallas as pl
from jax.experimental.pallas import tpu as pltpu
from jax.experimental.pallas import tpu_sc as plsc


_WIDE = (
    (25600, 400),
    (102400, 400),
    (236800, 400),
)
_CHUNK_ROWS = 1600
_NS = 12
_LAG = 6

_VSHAPE = (2080, 25, 50)
_VCHUNK = 260
_VN = _VSHAPE[0] // _VCHUNK

_CHUNKS = [(t, r0) for t, (rows, _) in enumerate(_WIDE)
           for r0 in range(0, rows, _CHUNK_ROWS)]
_VSTEPS = {12 + 12 * k: k for k in range(_VN)}


def _stream_body(s0, s1, s2, vx, o0, o1, o2, ov,
                 ring, vring, rsem, wsem, vrsem, vwsem):
    ins = (s0, s1, s2)
    outs = (o0, o1, o2)
    n = len(_CHUNKS)

    def rd(i):
        t, r0 = _CHUNKS[i]
        return pltpu.make_async_copy(
            ins[t].at[pl.ds(r0, _CHUNK_ROWS)], ring.at[i % _NS], rsem.at[i % _NS])

    def wr(i):
        t, r0 = _CHUNKS[i]
        return pltpu.make_async_copy(
            ring.at[i % _NS], outs[t].at[pl.ds(r0, _CHUNK_ROWS)], wsem.at[i % _NS])

    def vrd(k):
        return pltpu.make_async_copy(
            vx.at[pl.ds(k * _VCHUNK, _VCHUNK)], vring.at[k % 2], vrsem.at[k % 2])

    def vwr(k):
        return pltpu.make_async_copy(
            vring.at[k % 2], ov.at[pl.ds(k * _VCHUNK, _VCHUNK)], vwsem.at[k % 2])

    vrd(0).start()
    vrd(1).start()
    for i in range(n + _LAG):
        if i < n:
            if i >= _NS:
                wr(i - _NS).wait()
            rd(i).start()
        j = i - _LAG
        if 0 <= j < n:
            rd(j).wait()
            wr(j).start()
        k = _VSTEPS.get(i)
        if k is not None:
            if k >= 2:
                vwr(k - 2).wait()
            vrd(k).wait()
            vwr(k).start()
            if k + 2 < _VN:
                vrd(k + 2).start()
    for j in range(n - _NS, n):
        wr(j).wait()
    vwr(_VN - 2).wait()
    vwr(_VN - 1).wait()


def _tc_stream(wide, vx):
    return pl.pallas_call(
        _stream_body,
        in_specs=[pl.BlockSpec(memory_space=pl.ANY)] * 4,
        out_specs=[pl.BlockSpec(memory_space=pl.ANY)] * 4,
        out_shape=[jax.ShapeDtypeStruct(f.shape, f.dtype)
                   for f in (*wide, vx)],
        scratch_shapes=[
            pltpu.VMEM((_NS, _CHUNK_ROWS, 400), jnp.float32),
            pltpu.VMEM((2, _VCHUNK) + _VSHAPE[1:], jnp.float32),
            pltpu.SemaphoreType.DMA((_NS,)),
            pltpu.SemaphoreType.DMA((_NS,)),
            pltpu.SemaphoreType.DMA((2,)),
            pltpu.SemaphoreType.DMA((2,)),
        ],
        compiler_params=pltpu.CompilerParams(vmem_limit_bytes=60 * 1024 * 1024),
    )(*wide, vx)


_SC_ROWS = 19200
_SC_W = 400
_NWORK = 32
_RPW = _SC_ROWS // _NWORK
_SC_CHUNK = 120
_SC_NCH = _RPW // _SC_CHUNK


def _sc_copy_body(src, dst, buf0, buf1, sem0, sem1):
    wid = lax.axis_index("s") * 2 + lax.axis_index("c")
    base = wid * _RPW
    bufs = (buf0, buf1)
    sems = (sem0, sem1)

    def rd(k):
        return pltpu.make_async_copy(
            src.at[pl.ds(base + k * _SC_CHUNK, _SC_CHUNK)], bufs[k % 2], sems[k % 2])

    def wr(k):
        return pltpu.make_async_copy(
            bufs[k % 2], dst.at[pl.ds(base + k * _SC_CHUNK, _SC_CHUNK)], sems[k % 2])

    rd(0).start()
    rd(1).start()
    for k in range(_SC_NCH):
        rd(k).wait()
        wr(k).start()
        if k + 2 < _SC_NCH:
            wr(k).wait()
            rd(k + 2).start()
    wr(_SC_NCH - 2).wait()
    wr(_SC_NCH - 1).wait()


_sc_copy = functools.partial(
    pl.kernel,
    out_type=jax.ShapeDtypeStruct((_SC_ROWS, _SC_W), jnp.float32),
    mesh=plsc.VectorSubcoreMesh(core_axis_name="c", subcore_axis_name="s"),
    scratch_types=[
        pltpu.VMEM((_SC_CHUNK, _SC_W), jnp.float32),
        pltpu.VMEM((_SC_CHUNK, _SC_W), jnp.float32),
        pltpu.SemaphoreType.DMA,
        pltpu.SemaphoreType.DMA,
    ],
)(_sc_copy_body)


def kernel(semantic, distance, vertex, embedding, direction):
    wide = [semantic.reshape(_WIDE[0]), embedding.reshape(_WIDE[1]),
            direction.reshape(_WIDE[2])]
    vx = vertex.reshape(_VSHAPE)
    dis = _sc_copy(distance.reshape(_SC_ROWS, _SC_W))
    outs = _tc_stream(wide, vx)
    return (outs[0].reshape(semantic.shape), dis.reshape(distance.shape),
            outs[3].reshape(vertex.shape), outs[1].reshape(embedding.shape),
            outs[2].reshape(direction.shape))


# speedup vs baseline: 1.0010x; 1.0010x over previous
import functools

import jax
import jax.numpy as jnp
from jax import lax
from jax.experimental import pallas as pl
from jax.experimental.pallas import tpu as pltpu
from jax.experimental.pallas import tpu_sc as plsc


_WIDE = (
    (25600, 400),
    (102400, 400),
    (236800, 400),
)
_CHUNK_ROWS = 1600
_NS = 12
_LAG = 6

_VSHAPE = (2080, 25, 50)
_VCHUNK = 260
_VN = _VSHAPE[0] // _VCHUNK

_CHUNKS = [(t, r0) for t, (rows, _) in enumerate(_WIDE)
           for r0 in range(0, rows, _CHUNK_ROWS)]
_VSTEPS = {12 + 12 * k: k for k in range(_VN)}


def _stream_body(s0, s1, s2, vx, o0, o1, o2, ov,
                 ring, vring, rsem, wsem, vrsem, vwsem):
    ins = (s0, s1, s2)
    outs = (o0, o1, o2)
    n = len(_CHUNKS)

    def rd(i):
        t, r0 = _CHUNKS[i]
        return pltpu.make_async_copy(
            ins[t].at[pl.ds(r0, _CHUNK_ROWS)], ring.at[i % _NS], rsem.at[i % _NS])

    def wr(i):
        t, r0 = _CHUNKS[i]
        return pltpu.make_async_copy(
            ring.at[i % _NS], outs[t].at[pl.ds(r0, _CHUNK_ROWS)], wsem.at[i % _NS])

    def vrd(k):
        return pltpu.make_async_copy(
            vx.at[pl.ds(k * _VCHUNK, _VCHUNK)], vring.at[k % 2], vrsem.at[k % 2])

    def vwr(k):
        return pltpu.make_async_copy(
            vring.at[k % 2], ov.at[pl.ds(k * _VCHUNK, _VCHUNK)], vwsem.at[k % 2])

    vrd(0).start()
    vrd(1).start()
    for i in range(n + _LAG):
        if i < n:
            if i >= _NS:
                wr(i - _NS).wait()
            rd(i).start()
        j = i - _LAG
        if 0 <= j < n:
            rd(j).wait()
            wr(j).start()
        k = _VSTEPS.get(i)
        if k is not None:
            if k >= 2:
                vwr(k - 2).wait()
            vrd(k).wait()
            vwr(k).start()
            if k + 2 < _VN:
                vrd(k + 2).start()
    for j in range(n - _NS, n):
        wr(j).wait()
    vwr(_VN - 2).wait()
    vwr(_VN - 1).wait()


def _tc_stream(wide, vx):
    return pl.pallas_call(
        _stream_body,
        in_specs=[pl.BlockSpec(memory_space=pl.ANY)] * 4,
        out_specs=[pl.BlockSpec(memory_space=pl.ANY)] * 4,
        out_shape=[jax.ShapeDtypeStruct(f.shape, f.dtype)
                   for f in (*wide, vx)],
        scratch_shapes=[
            pltpu.VMEM((_NS, _CHUNK_ROWS, 400), jnp.float32),
            pltpu.VMEM((2, _VCHUNK) + _VSHAPE[1:], jnp.float32),
            pltpu.SemaphoreType.DMA((_NS,)),
            pltpu.SemaphoreType.DMA((_NS,)),
            pltpu.SemaphoreType.DMA((2,)),
            pltpu.SemaphoreType.DMA((2,)),
        ],
        compiler_params=pltpu.CompilerParams(vmem_limit_bytes=60 * 1024 * 1024),
    )(*wide, vx)


_SC_ROWS = 19200
_SC_W = 400
_NWORK = 32
_RPW = _SC_ROWS // _NWORK
_SC_CHUNK = 120
_SC_NCH = _RPW // _SC_CHUNK


def _sc_copy_body(src, dst, buf0, buf1, sem0, sem1):
    wid = lax.axis_index("s") * 2 + lax.axis_index("c")
    base = wid * _RPW
    bufs = (buf0, buf1)
    sems = (sem0, sem1)

    def rd(k):
        return pltpu.make_async_copy(
            src.at[pl.ds(base + k * _SC_CHUNK, _SC_CHUNK)], bufs[k % 2], sems[k % 2])

    def wr(k):
        return pltpu.make_async_copy(
            bufs[k % 2], dst.at[pl.ds(base + k * _SC_CHUNK, _SC_CHUNK)], sems[k % 2])

    rd(0).start()
    rd(1).start()
    for k in range(_SC_NCH):
        rd(k).wait()
        wr(k).start()
        if k + 2 < _SC_NCH:
            wr(k).wait()
            rd(k + 2).start()
    wr(_SC_NCH - 2).wait()
    wr(_SC_NCH - 1).wait()


_sc_copy = functools.partial(
    pl.kernel,
    out_type=jax.ShapeDtypeStruct((_SC_ROWS, _SC_W), jnp.float32),
    mesh=plsc.VectorSubcoreMesh(core_axis_name="c", subcore_axis_name="s"),
    scratch_types=[
        pltpu.VMEM((_SC_CHUNK, _SC_W), jnp.float32),
        pltpu.VMEM((_SC_CHUNK, _SC_W), jnp.float32),
        pltpu.SemaphoreType.DMA,
        pltpu.SemaphoreType.DMA,
    ],
    compiler_params=pltpu.CompilerParams(use_tc_tiling_on_sc=True),
)(_sc_copy_body)


def kernel(semantic, distance, vertex, embedding, direction):
    wide = [semantic.reshape(_WIDE[0]), embedding.reshape(_WIDE[1]),
            direction.reshape(_WIDE[2])]
    vx = vertex.reshape(_VSHAPE)
    dis = _sc_copy(distance.reshape(_SC_ROWS, _SC_W))
    outs = _tc_stream(wide, vx)
    return (outs[0].reshape(semantic.shape), dis.reshape(distance.shape),
            outs[3].reshape(vertex.shape), outs[1].reshape(embedding.shape),
            outs[2].reshape(direction.shape))


# speedup vs baseline: 1.0120x; 1.0110x over previous
import jax
import jax.numpy as jnp
from jax.experimental import pallas as pl
from jax.experimental.pallas import tpu as pltpu

_WIDE = (
    (25600, 400),
    (19200, 400),
    (102400, 400),
    (236800, 400),
)
_MAIN_W = 384
_TAIL_W = 16
_CHUNK_ROWS = 1600
_NS = 14
_LAG = 7

_TCHUNK_ROWS = 6400
_NT = 4
_TLAG = 2

_VSHAPE = (2080, 25, 50)
_VCHUNK = 260
_VN = _VSHAPE[0] // _VCHUNK

_CHUNKS = [(t, r0) for t, (rows, _) in enumerate(_WIDE)
           for r0 in range(0, rows, _CHUNK_ROWS)]
_TCHUNKS = [(t, r0) for t, (rows, _) in enumerate(_WIDE)
            for r0 in range(0, rows, _TCHUNK_ROWS)]
_TSTEPS = {3 + 4 * k: k for k in range(len(_TCHUNKS))}
_VSTEPS = {12 + 12 * k: k for k in range(_VN)}


def _stream_body(s0, s1, s2, s3, vx, o0, o1, o2, o3, ov,
                 ring, tring, vring, rsem, wsem, trsem, twsem, vrsem, vwsem):
    ins = (s0, s1, s2, s3)
    outs = (o0, o1, o2, o3)
    n = len(_CHUNKS)

    def rd(i):
        t, r0 = _CHUNKS[i]
        return pltpu.make_async_copy(
            ins[t].at[pl.ds(r0, _CHUNK_ROWS), pl.ds(0, _MAIN_W)],
            ring.at[i % _NS], rsem.at[i % _NS])

    def wr(i):
        t, r0 = _CHUNKS[i]
        return pltpu.make_async_copy(
            ring.at[i % _NS],
            outs[t].at[pl.ds(r0, _CHUNK_ROWS), pl.ds(0, _MAIN_W)],
            wsem.at[i % _NS])

    def trd(i):
        t, r0 = _TCHUNKS[i]
        return pltpu.make_async_copy(
            ins[t].at[pl.ds(r0, _TCHUNK_ROWS), pl.ds(_MAIN_W, _TAIL_W)],
            tring.at[i % _NT], trsem.at[i % _NT])

    def twr(i):
        t, r0 = _TCHUNKS[i]
        return pltpu.make_async_copy(
            tring.at[i % _NT],
            outs[t].at[pl.ds(r0, _TCHUNK_ROWS), pl.ds(_MAIN_W, _TAIL_W)],
            twsem.at[i % _NT])

    def vrd(k):
        return pltpu.make_async_copy(
            vx.at[pl.ds(k * _VCHUNK, _VCHUNK)], vring.at[k % 2], vrsem.at[k % 2])

    def vwr(k):
        return pltpu.make_async_copy(
            vring.at[k % 2], ov.at[pl.ds(k * _VCHUNK, _VCHUNK)], vwsem.at[k % 2])

    nt = len(_TCHUNKS)
    vrd(0).start()
    vrd(1).start()
    for i in range(_TLAG):
        trd(i).start()
    for i in range(n + _LAG):
        if i < n:
            if i >= _NS:
                wr(i - _NS).wait()
            rd(i).start()
        j = i - _LAG
        if 0 <= j < n:
            rd(j).wait()
            wr(j).start()
        tk = _TSTEPS.get(i)
        if tk is not None:
            if tk >= _NT:
                twr(tk - _NT).wait()
            trd(tk).wait()
            twr(tk).start()
            if tk + _TLAG < nt:
                trd(tk + _TLAG).start()
        k = _VSTEPS.get(i)
        if k is not None:
            if k >= 2:
                vwr(k - 2).wait()
            vrd(k).wait()
            vwr(k).start()
            if k + 2 < _VN:
                vrd(k + 2).start()
    for j in range(n - _NS, n):
        wr(j).wait()
    for j in range(nt - _NT, nt):
        twr(j).wait()
    vwr(_VN - 2).wait()
    vwr(_VN - 1).wait()


def kernel(semantic, distance, vertex, embedding, direction):
    wide = [semantic.reshape(_WIDE[0]), distance.reshape(_WIDE[1]),
            embedding.reshape(_WIDE[2]), direction.reshape(_WIDE[3])]
    vx = vertex.reshape(_VSHAPE)
    outs = pl.pallas_call(
        _stream_body,
        in_specs=[pl.BlockSpec(memory_space=pl.ANY)] * 5,
        out_specs=[pl.BlockSpec(memory_space=pl.ANY)] * 5,
        out_shape=[jax.ShapeDtypeStruct(f.shape, f.dtype)
                   for f in (*wide, vx)],
        scratch_shapes=[
            pltpu.VMEM((_NS, _CHUNK_ROWS, _MAIN_W), jnp.float32),
            pltpu.VMEM((_NT, _TCHUNK_ROWS, _TAIL_W), jnp.float32),
            pltpu.VMEM((2, _VCHUNK) + _VSHAPE[1:], jnp.float32),
            pltpu.SemaphoreType.DMA((_NS,)),
            pltpu.SemaphoreType.DMA((_NS,)),
            pltpu.SemaphoreType.DMA((_NT,)),
            pltpu.SemaphoreType.DMA((_NT,)),
            pltpu.SemaphoreType.DMA((2,)),
            pltpu.SemaphoreType.DMA((2,)),
        ],
        compiler_params=pltpu.CompilerParams(vmem_limit_bytes=60 * 1024 * 1024),
    )(*wide, vx)
    return (outs[0].reshape(semantic.shape), outs[1].reshape(distance.shape),
            outs[4].reshape(vertex.shape), outs[2].reshape(embedding.shape),
            outs[3].reshape(direction.shape))


# speedup vs baseline: 1.1469x; 1.1333x over previous
import jax
import jax.numpy as jnp
from jax.experimental import pallas as pl
from jax.experimental.pallas import tpu as pltpu

_WIDE = (
    (25600, 400),
    (19200, 400),
    (102400, 400),
    (236800, 400),
)
_CHUNK_ROWS = 1600
_NS = 14
_LAG = 7

_VSHAPE = (1250, 32, 65)
_VCHUNK = 125
_VN = _VSHAPE[0] // _VCHUNK

_CHUNKS = [(t, r0) for t, (rows, _) in enumerate(_WIDE)
           for r0 in range(0, rows, _CHUNK_ROWS)]
_VSTEPS = {12 + 12 * k: k for k in range(_VN)}


def _stream_body(s0, s1, s2, s3, vx, o0, o1, o2, o3, ov,
                 ring, vring, rsem, wsem, vrsem, vwsem):
    ins = (s0, s1, s2, s3)
    outs = (o0, o1, o2, o3)
    n = len(_CHUNKS)

    def rd(i):
        t, r0 = _CHUNKS[i]
        return pltpu.make_async_copy(
            ins[t].at[pl.ds(r0, _CHUNK_ROWS)], ring.at[i % _NS], rsem.at[i % _NS])

    def wr(i):
        t, r0 = _CHUNKS[i]
        return pltpu.make_async_copy(
            ring.at[i % _NS], outs[t].at[pl.ds(r0, _CHUNK_ROWS)], wsem.at[i % _NS])

    def vrd(k):
        return pltpu.make_async_copy(
            vx.at[pl.ds(k * _VCHUNK, _VCHUNK)], vring.at[k % 2], vrsem.at[k % 2])

    def vwr(k):
        return pltpu.make_async_copy(
            vring.at[k % 2], ov.at[pl.ds(k * _VCHUNK, _VCHUNK)], vwsem.at[k % 2])

    vrd(0).start()
    vrd(1).start()
    for i in range(n + _LAG):
        if i < n:
            if i >= _NS:
                wr(i - _NS).wait()
            rd(i).start()
        j = i - _LAG
        if 0 <= j < n:
            rd(j).wait()
            wr(j).start()
        k = _VSTEPS.get(i)
        if k is not None:
            if k >= 2:
                vwr(k - 2).wait()
            vrd(k).wait()
            vwr(k).start()
            if k + 2 < _VN:
                vrd(k + 2).start()
    for j in range(n - _NS, n):
        wr(j).wait()
    vwr(_VN - 2).wait()
    vwr(_VN - 1).wait()


def kernel(semantic, distance, vertex, embedding, direction):
    wide = [semantic.reshape(_WIDE[0]), distance.reshape(_WIDE[1]),
            embedding.reshape(_WIDE[2]), direction.reshape(_WIDE[3])]
    vx = vertex.transpose(2, 3, 0, 1).reshape(_VSHAPE)
    outs = pl.pallas_call(
        _stream_body,
        in_specs=[pl.BlockSpec(memory_space=pl.ANY)] * 5,
        out_specs=[pl.BlockSpec(memory_space=pl.ANY)] * 5,
        out_shape=[jax.ShapeDtypeStruct(f.shape, f.dtype)
                   for f in (*wide, vx)],
        scratch_shapes=[
            pltpu.VMEM((_NS, _CHUNK_ROWS, 400), jnp.float32),
            pltpu.VMEM((2, _VCHUNK) + _VSHAPE[1:], jnp.float32),
            pltpu.SemaphoreType.DMA((_NS,)),
            pltpu.SemaphoreType.DMA((_NS,)),
            pltpu.SemaphoreType.DMA((2,)),
            pltpu.SemaphoreType.DMA((2,)),
        ],
        compiler_params=pltpu.CompilerParams(vmem_limit_bytes=60 * 1024 * 1024),
    )(*wide, vx)
    ver = outs[4].reshape(25, 50, 32, 65).transpose(2, 3, 0, 1)
    return (outs[0].reshape(semantic.shape), outs[1].reshape(distance.shape),
            ver, outs[2].reshape(embedding.shape),
            outs[3].reshape(direction.shape))
